# baseline (speedup 1.0000x reference)
import functools

import jax
import jax.numpy as jnp
from jax import lax
from jax.experimental import pallas as pl
from jax.experimental.pallas import tpu as pltpu
from jax.experimental.pallas import tpu_sc as plsc

P_TOTAL = 131072
D = 256
N_HEADS = 8
N_BLOCKS = 256
N_ATTR = 65536
NC, NS = 2, 16
HALF = P_TOTAL // NC
P_TILE = HALF // NS
WIN = 128
NWIN = P_TILE // WIN
A_TILE = N_ATTR // NS
ZROWS = 128
TBL_ROWS = N_HEADS * N_BLOCKS + ZROWS
TRASH = 2048
CNT = N_HEADS * HALF
FSLOTS = 16
APC = 1024

_mesh = plsc.VectorSubcoreMesh(core_axis_name="c", subcore_axis_name="s")


@functools.partial(
    pl.kernel,
    mesh=_mesh,
    out_type=jax.ShapeDtypeStruct((P_TOTAL, D), jnp.float32),
    scratch_types=[
        pltpu.VMEM((APC // 128, 128), jnp.int32),
        pltpu.VMEM((128,), jnp.int32),
        pltpu.VMEM((1024,), jnp.int32),
        pltpu.VMEM((P_TILE,), jnp.int32),
        pltpu.VMEM((N_HEADS, WIN), jnp.int32),
        pltpu.VMEM((N_HEADS, WIN), jnp.int32),
        pltpu.VMEM((WIN,), jnp.int32),
        pltpu.VMEM((WIN,), jnp.int32),
        pltpu.VMEM((N_HEADS * 16,), jnp.int32),
        pltpu.VMEM((N_HEADS * 16,), jnp.int32),
        pltpu.VMEM((16,), jnp.int32),
        pltpu.VMEM((16,), jnp.int32),
        pltpu.VMEM((WIN, D), jnp.float32),
        pltpu.VMEM((WIN, D), jnp.float32),
        pltpu.VMEM((FSLOTS, D), jnp.float32),
        pltpu.VMEM((FSLOTS * 16,), jnp.int32),
        pltpu.VMEM((FSLOTS * 16,), jnp.float32),
        pltpu.VMEM((16,), jnp.int32),
        pltpu.VMEM_SHARED((CNT + TRASH,), jnp.int32),
        pltpu.SemaphoreType.DMA,
        pltpu.SemaphoreType.DMA,
        pltpu.SemaphoreType.DMA,
        pltpu.SemaphoreType.DMA,
        pltpu.SemaphoreType.DMA,
        pltpu.SemaphoreType.DMA,
        pltpu.SemaphoreType.DMA,
    ],
)
def _encode(grid_h, pos_h, head_h, tbl_h, out_h,
            sidx, ones_v, zeros_v, grid_all,
            cntA, cntB, idxA, idxB, indA, indB, ntotA, ntotB, winA, winB,
            rowfs, pploc, pscale, smcv, cnt_s,
            sem_a, sem_g0, sem_g1, sem_o0, sem_o1, sem_r, sem_c):
    c = lax.axis_index("c")
    s = lax.axis_index("s")
    half_base = c * HALF
    tile_pbase = half_base + s * P_TILE
    lanes = lax.broadcasted_iota(jnp.int32, (16,), 0)

    for g in range(128 // 16):
        ones_v[pl.ds(g * 16, 16)] = jnp.ones((16,), jnp.int32)

    def _zinit(g, carry):
        zeros_v[pl.ds(g * 16, 16)] = jnp.zeros((16,), jnp.int32)
        return carry
    lax.fori_loop(0, 1024 // 16, _zinit, 0)

    def _ziss(j, carry):
        pltpu.async_copy(zeros_v,
                         cnt_s.at[pl.ds(s * (CNT // NS) + j * 1024, 1024)],
                         sem_a)
        return carry
    lax.fori_loop(0, 32, _ziss, 0)

    def _zdrain(j, carry):
        pltpu.make_async_copy(
            zeros_v, cnt_s.at[pl.ds(s * (CNT // NS), 1024)], sem_a).wait()
        return carry
    lax.fori_loop(0, 32, _zdrain, 0)

    plsc.subcore_barrier()

    def _chunk(q, carry):
        pltpu.sync_copy(pos_h.at[pl.ds(s * A_TILE + q * APC, APC)],
                        grid_all.at[pl.ds(0, APC)])
        pltpu.sync_copy(head_h.at[pl.ds(s * A_TILE + q * APC, APC)],
                        grid_all.at[pl.ds(APC, APC)])

        def _bidx(g, carry2):
            p = grid_all[pl.ds(g * 16, 16)]
            h = grid_all[pl.ds(APC + g * 16, 16)]
            rel = p - half_base
            ok = (rel >= 0) & (rel < HALF)
            tgt = h * HALF + rel
            spread = (q * APC + g * 16 + lanes) & (TRASH - 1)
            idx = jnp.where(ok, tgt, CNT + spread)
            row = g // 8
            col = (g % 8) * 16
            sidx[row, pl.ds(col, 16)] = idx
            return carry2
        lax.fori_loop(0, APC // 16, _bidx, 0)

        def _hist(j, carry2):
            pltpu.sync_copy(ones_v, cnt_s.at[sidx.at[j]], add=True)
            return carry2
        lax.fori_loop(0, APC // 128, _hist, 0)
        return carry
    lax.fori_loop(0, A_TILE // APC, _chunk, 0)
    pltpu.sync_copy(grid_h.at[pl.ds(tile_pbase, P_TILE)], grid_all)

    plsc.subcore_barrier()

    def _plan(wq, cntb, idxb, indb, ntb):
        lbase = s * P_TILE + wq * WIN
        for h in range(N_HEADS):
            pltpu.async_copy(cnt_s.at[pl.ds(h * HALF + lbase, WIN)],
                             cntb.at[h], sem_c)
        for h in range(N_HEADS):
            pltpu.make_async_copy(cnt_s.at[pl.ds(0, WIN)],
                                  cntb.at[h], sem_c).wait()
        ind = [jnp.zeros((16,), jnp.int32) for _ in range(N_HEADS)]
        nfix = jnp.zeros((16,), jnp.int32)
        for g in range(WIN // 16):
            b = grid_all[pl.ds(wq * WIN + g * 16, 16)]
            first = N_HEADS * N_BLOCKS + ((g * 16 + lanes) & 15)
            nsel = jnp.zeros((16,), jnp.int32)
            for h in range(N_HEADS):
                cv = cntb[h, pl.ds(g * 16, 16)]
                has = cv > 0
                isfirst = has & (nsel == 0)
                rowi = h * N_BLOCKS + b
                first = jnp.where(isfirst, rowi, first)
                bA = isfirst & (cv > 1)
                bB = has & (nsel > 0)
                ind[h] = (ind[h]
                          | jnp.where(bB, 1 << g, 0)
                          | jnp.where(bA, 1 << (8 + g), 0))
                nfix = (nfix + jnp.where(bA, 1, 0) + jnp.where(bB, 1, 0))
                nsel = nsel + jnp.where(has, 1, 0)
            idxb[pl.ds(g * 16, 16)] = first
        for h in range(N_HEADS):
            indb[pl.ds(h * 16, 16)] = ind[h]
        ntb[pl.ds(0, 16)] = nfix

    def _walkrec(w, cntb, indb, r16):
        smcv[pl.ds(0, 16)] = jnp.zeros((16,), jnp.int32)

        def _wh(h, carry):
            mv = indb[pl.ds(h * 16, 16)]
            for l in range(16):
                m0 = mv[l]

                @pl.when(m0 != 0)
                def _(m0=m0, l=l):
                    def _g2(g2, cc):
                        @pl.when(((m0 >> g2) & 1) != 0)
                        def _():
                            nf = smcv[pl.ds(0, 16)][0]

                            @pl.when((nf >= r16) & (nf < r16 + FSLOTS))
                            def _():
                                isA = g2 // 8
                                g = g2 & 7
                                b = grid_all[pl.ds(w * WIN + g * 16,
                                                   16)][l]
                                rowi = h * N_BLOCKS + b
                                cval = cntb[h, pl.ds(g * 16, 16)][l]
                                scale = (cval - isA).astype(jnp.float32)
                                slot = nf - r16
                                pploc[pl.ds(slot * 16, 16)] = (
                                    jnp.broadcast_to(g * 16 + l, (16,)))
                                pscale[pl.ds(slot * 16, 16)] = (
                                    jnp.broadcast_to(scale, (16,)))
                                pltpu.async_copy(
                                    tbl_h.at[rowi], rowfs.at[slot], sem_r)
                            smcv[pl.ds(0, 16)] = jnp.broadcast_to(
                                nf + 1, (16,))
                        return cc
                    lax.fori_loop(0, 16, _g2, 0)
            return carry
        lax.fori_loop(0, N_HEADS, _wh, 0)

    def _rdrain(n):
        def _rd(j, carry):
            pltpu.make_async_copy(tbl_h.at[0], rowfs.at[0], sem_r).wait()
            return carry
        lax.fori_loop(0, n, _rd, 0)

    def _fixups(w, winb, cntb, indb, ntb):
        nv = ntb[pl.ds(0, 16)]
        total = nv[0]
        for l in range(1, 16):
            total = total + nv[l]

        def _round(r, carry):
            r16 = r * FSLOTS
            nret = jnp.minimum(total - r16, FSLOTS)
            _walkrec(w, cntb, indb, r16)
            _rdrain(nret)

            def _apply(j, c2):
                ploc = pploc[pl.ds(j * 16, 16)][0]
                scv = jnp.broadcast_to(pscale[pl.ds(j * 16, 16)][0], (16,))

                def _fma(d, c3):
                    winb[ploc, pl.ds(d * 16, 16)] = (
                        winb[ploc, pl.ds(d * 16, 16)]
                        + scv * rowfs[j, pl.ds(d * 16, 16)])
                    return c3
                lax.fori_loop(0, D // 16, _fma, 0)
                return c2
            lax.fori_loop(0, nret, _apply, 0)
            return carry
        lax.fori_loop(0, (total + FSLOTS - 1) // FSLOTS, _round, 0)

    _plan(jnp.int32(0), cntA, idxA, indA, ntotA)
    pltpu.async_copy(tbl_h.at[idxA], winA, sem_g0)

    def _pair(k, carry):
        w0 = 2 * k
        w1 = w0 + 1

        _plan(w1, cntB, idxB, indB, ntotB)

        @pl.when(w0 >= 1)
        def _():
            pltpu.make_async_copy(winB, out_h.at[pl.ds(0, WIN)],
                                  sem_o1).wait()
        pltpu.async_copy(tbl_h.at[idxB], winB, sem_g1)
        pltpu.make_async_copy(tbl_h.at[idxA], winA, sem_g0).wait()
        _fixups(w0, winA, cntA, indA, ntotA)
        pltpu.async_copy(winA, out_h.at[pl.ds(tile_pbase + w0 * WIN, WIN)],
                         sem_o0)

        @pl.when(w1 + 1 < NWIN)
        def _():
            _plan(w1 + 1, cntA, idxA, indA, ntotA)
        pltpu.make_async_copy(winA, out_h.at[pl.ds(0, WIN)], sem_o0).wait()

        @pl.when(w1 + 1 < NWIN)
        def _():
            pltpu.async_copy(tbl_h.at[idxA], winA, sem_g0)
        pltpu.make_async_copy(tbl_h.at[idxB], winB, sem_g1).wait()
        _fixups(w1, winB, cntB, indB, ntotB)
        pltpu.async_copy(winB, out_h.at[pl.ds(tile_pbase + w1 * WIN, WIN)],
                         sem_o1)
        return carry
    lax.fori_loop(0, NWIN // 2, _pair, 0)

    pltpu.make_async_copy(winB, out_h.at[pl.ds(0, WIN)], sem_o1).wait()


@jax.jit
def _run(grid_flat, attr_positions, attr_head_ids, table_ext):
    return _encode(grid_flat, attr_positions, attr_head_ids, table_ext)


def kernel(block_type_grid, attr_positions, attr_head_ids, heads):
    Bt, Wt, Ht, Lt = block_type_grid.shape
    grid_flat = block_type_grid.reshape(-1)
    table_ext = jnp.concatenate(
        [heads.reshape(N_HEADS * N_BLOCKS, D),
         jnp.zeros((ZROWS, D), heads.dtype)], axis=0)
    out = _run(grid_flat, attr_positions, attr_head_ids, table_ext)
    return out.reshape(Bt, Wt, Ht, Lt, D)


# speedup vs baseline: 1.2993x; 1.2993x over previous
import functools

import jax
import jax.numpy as jnp
from jax import lax
from jax.experimental import pallas as pl
from jax.experimental.pallas import tpu as pltpu
from jax.experimental.pallas import tpu_sc as plsc

P_TOTAL = 131072
D = 256
N_HEADS = 8
N_BLOCKS = 256
N_ATTR = 65536
NC, NS = 2, 16
HALF = P_TOTAL // NC
P_TILE = HALF // NS
WIN = 128
NWIN = P_TILE // WIN
A_TILE = N_ATTR // NS
ZROWS = 128
TBL_ROWS = N_HEADS * N_BLOCKS + ZROWS
TRASH = 2048
CNT = N_HEADS * HALF
FSLOTS = 16
APC = 1024

_mesh = plsc.VectorSubcoreMesh(core_axis_name="c", subcore_axis_name="s")


@functools.partial(
    pl.kernel,
    mesh=_mesh,
    out_type=jax.ShapeDtypeStruct((P_TOTAL, D), jnp.float32),
    scratch_types=[
        pltpu.VMEM((APC // 128, 128), jnp.int32),
        pltpu.VMEM((128,), jnp.int32),
        pltpu.VMEM((1024,), jnp.int32),
        pltpu.VMEM((P_TILE,), jnp.int32),
        pltpu.VMEM((N_HEADS, WIN), jnp.int32),
        pltpu.VMEM((N_HEADS, WIN), jnp.int32),
        pltpu.VMEM((WIN,), jnp.int32),
        pltpu.VMEM((WIN,), jnp.int32),
        pltpu.VMEM((N_HEADS * 16,), jnp.int32),
        pltpu.VMEM((N_HEADS * 16,), jnp.int32),
        pltpu.VMEM((16,), jnp.int32),
        pltpu.VMEM((16,), jnp.int32),
        pltpu.VMEM((WIN, D), jnp.float32),
        pltpu.VMEM((WIN, D), jnp.float32),
        pltpu.VMEM((FSLOTS, D), jnp.float32),
        pltpu.VMEM((FSLOTS * 16,), jnp.int32),
        pltpu.VMEM((FSLOTS * 16,), jnp.float32),
        pltpu.VMEM((16,), jnp.int32),
        pltpu.VMEM_SHARED((CNT + TRASH,), jnp.int32),
        pltpu.SemaphoreType.DMA,
        pltpu.SemaphoreType.DMA,
        pltpu.SemaphoreType.DMA,
        pltpu.SemaphoreType.DMA,
        pltpu.SemaphoreType.DMA,
        pltpu.SemaphoreType.DMA,
        pltpu.SemaphoreType.DMA,
    ],
)
def _encode(grid_h, pos_h, head_h, tbl_h, out_h,
            sidx, ones_v, zeros_v, grid_all,
            cntA, cntB, idxA, idxB, indA, indB, ntotA, ntotB, winA, winB,
            rowfs, pploc, pscale, smcv, cnt_s,
            sem_a, sem_g0, sem_g1, sem_o0, sem_o1, sem_r, sem_c):
    c = lax.axis_index("c")
    s = lax.axis_index("s")
    half_base = c * HALF
    tile_pbase = half_base + s * P_TILE
    lanes = lax.broadcasted_iota(jnp.int32, (16,), 0)

    for g in range(128 // 16):
        ones_v[pl.ds(g * 16, 16)] = jnp.ones((16,), jnp.int32)

    def _zinit(g, carry):
        zeros_v[pl.ds(g * 16, 16)] = jnp.zeros((16,), jnp.int32)
        return carry
    lax.fori_loop(0, 1024 // 16, _zinit, 0)

    def _ziss(j, carry):
        pltpu.async_copy(zeros_v,
                         cnt_s.at[pl.ds(s * (CNT // NS) + j * 1024, 1024)],
                         sem_a)
        return carry
    lax.fori_loop(0, 32, _ziss, 0)

    def _zdrain(j, carry):
        pltpu.make_async_copy(
            zeros_v, cnt_s.at[pl.ds(s * (CNT // NS), 1024)], sem_a).wait()
        return carry
    lax.fori_loop(0, 32, _zdrain, 0)

    plsc.subcore_barrier()

    def _chunk(q, carry):
        pltpu.sync_copy(pos_h.at[pl.ds(s * A_TILE + q * APC, APC)],
                        grid_all.at[pl.ds(0, APC)])
        pltpu.sync_copy(head_h.at[pl.ds(s * A_TILE + q * APC, APC)],
                        grid_all.at[pl.ds(APC, APC)])

        def _bidx(g, carry2):
            p = grid_all[pl.ds(g * 16, 16)]
            h = grid_all[pl.ds(APC + g * 16, 16)]
            rel = p - half_base
            ok = (rel >= 0) & (rel < HALF)
            tgt = h * HALF + rel
            spread = (q * APC + g * 16 + lanes) & (TRASH - 1)
            idx = jnp.where(ok, tgt, CNT + spread)
            row = g // 8
            col = (g % 8) * 16
            sidx[row, pl.ds(col, 16)] = idx
            return carry2
        lax.fori_loop(0, APC // 16, _bidx, 0)

        def _hiss(j, carry2):
            pltpu.async_copy(ones_v, cnt_s.at[sidx.at[j]], sem_a, add=True)
            return carry2
        lax.fori_loop(0, APC // 128, _hiss, 0)

        def _hdrain(j, carry2):
            pltpu.make_async_copy(ones_v, cnt_s.at[sidx.at[0]],
                                  sem_a).wait()
            return carry2
        lax.fori_loop(0, APC // 128, _hdrain, 0)
        return carry
    lax.fori_loop(0, A_TILE // APC, _chunk, 0)
    pltpu.sync_copy(grid_h.at[pl.ds(tile_pbase, P_TILE)], grid_all)

    plsc.subcore_barrier()

    def _plan(wq, cntb, idxb, indb, ntb):
        lbase = s * P_TILE + wq * WIN
        for h in range(N_HEADS):
            pltpu.async_copy(cnt_s.at[pl.ds(h * HALF + lbase, WIN)],
                             cntb.at[h], sem_c)
        for h in range(N_HEADS):
            pltpu.make_async_copy(cnt_s.at[pl.ds(0, WIN)],
                                  cntb.at[h], sem_c).wait()
        ind = [jnp.zeros((16,), jnp.int32) for _ in range(N_HEADS)]
        nfix = jnp.zeros((16,), jnp.int32)
        for g in range(WIN // 16):
            b = grid_all[pl.ds(wq * WIN + g * 16, 16)]
            first = N_HEADS * N_BLOCKS + g * 16 + lanes
            nsel = jnp.zeros((16,), jnp.int32)
            for h in range(N_HEADS):
                cv = cntb[h, pl.ds(g * 16, 16)]
                has = cv > 0
                isfirst = has & (nsel == 0)
                rowi = h * N_BLOCKS + b
                first = jnp.where(isfirst, rowi, first)
                bA = isfirst & (cv > 1)
                bB = has & (nsel > 0)
                ind[h] = (ind[h]
                          | jnp.where(bB, 1 << g, 0)
                          | jnp.where(bA, 1 << (8 + g), 0))
                nfix = (nfix + jnp.where(bA, 1, 0) + jnp.where(bB, 1, 0))
                nsel = nsel + jnp.where(has, 1, 0)
            idxb[pl.ds(g * 16, 16)] = first
        for h in range(N_HEADS):
            indb[pl.ds(h * 16, 16)] = ind[h]
        ntb[pl.ds(0, 16)] = nfix

    def _walkrec(w, cntb, indb, r16):
        smcv[pl.ds(0, 16)] = jnp.zeros((16,), jnp.int32)

        def _wh(h, carry):
            mv = indb[pl.ds(h * 16, 16)]
            for l in range(16):
                m0 = mv[l]

                @pl.when(m0 != 0)
                def _(m0=m0, l=l):
                    def _g2(g2, cc):
                        @pl.when(((m0 >> g2) & 1) != 0)
                        def _():
                            nf = smcv[pl.ds(0, 16)][0]

                            @pl.when((nf >= r16) & (nf < r16 + FSLOTS))
                            def _():
                                isA = g2 // 8
                                g = g2 & 7
                                b = grid_all[pl.ds(w * WIN + g * 16,
                                                   16)][l]
                                rowi = h * N_BLOCKS + b
                                cval = cntb[h, pl.ds(g * 16, 16)][l]
                                scale = (cval - isA).astype(jnp.float32)
                                slot = nf - r16
                                pploc[pl.ds(slot * 16, 16)] = (
                                    jnp.broadcast_to(g * 16 + l, (16,)))
                                pscale[pl.ds(slot * 16, 16)] = (
                                    jnp.broadcast_to(scale, (16,)))
                                pltpu.async_copy(
                                    tbl_h.at[rowi], rowfs.at[slot], sem_r)
                            smcv[pl.ds(0, 16)] = jnp.broadcast_to(
                                nf + 1, (16,))
                        return cc
                    lax.fori_loop(0, 16, _g2, 0)
            return carry
        lax.fori_loop(0, N_HEADS, _wh, 0)

    def _rdrain(n):
        def _rd(j, carry):
            pltpu.make_async_copy(tbl_h.at[0], rowfs.at[0], sem_r).wait()
            return carry
        lax.fori_loop(0, n, _rd, 0)

    def _fixups(w, winb, cntb, indb, ntb):
        nv = ntb[pl.ds(0, 16)]
        total = nv[0]
        for l in range(1, 16):
            total = total + nv[l]

        def _round(r, carry):
            r16 = r * FSLOTS
            nret = jnp.minimum(total - r16, FSLOTS)
            _walkrec(w, cntb, indb, r16)
            _rdrain(nret)

            def _apply(j, c2):
                ploc = pploc[pl.ds(j * 16, 16)][0]
                scv = jnp.broadcast_to(pscale[pl.ds(j * 16, 16)][0], (16,))

                def _fma(d, c3):
                    winb[ploc, pl.ds(d * 16, 16)] = (
                        winb[ploc, pl.ds(d * 16, 16)]
                        + scv * rowfs[j, pl.ds(d * 16, 16)])
                    return c3
                lax.fori_loop(0, D // 16, _fma, 0)
                return c2
            lax.fori_loop(0, nret, _apply, 0)
            return carry
        lax.fori_loop(0, (total + FSLOTS - 1) // FSLOTS, _round, 0)

    _plan(jnp.int32(0), cntA, idxA, indA, ntotA)
    pltpu.async_copy(tbl_h.at[idxA], winA, sem_g0)

    def _pair(k, carry):
        w0 = 2 * k
        w1 = w0 + 1

        _plan(w1, cntB, idxB, indB, ntotB)

        @pl.when(w0 >= 1)
        def _():
            pltpu.make_async_copy(winB, out_h.at[pl.ds(0, WIN)],
                                  sem_o1).wait()
        pltpu.async_copy(tbl_h.at[idxB], winB, sem_g1)
        pltpu.make_async_copy(tbl_h.at[idxA], winA, sem_g0).wait()
        _fixups(w0, winA, cntA, indA, ntotA)
        pltpu.async_copy(winA, out_h.at[pl.ds(tile_pbase + w0 * WIN, WIN)],
                         sem_o0)

        @pl.when(w1 + 1 < NWIN)
        def _():
            _plan(w1 + 1, cntA, idxA, indA, ntotA)
        pltpu.make_async_copy(winA, out_h.at[pl.ds(0, WIN)], sem_o0).wait()

        @pl.when(w1 + 1 < NWIN)
        def _():
            pltpu.async_copy(tbl_h.at[idxA], winA, sem_g0)
        pltpu.make_async_copy(tbl_h.at[idxB], winB, sem_g1).wait()
        _fixups(w1, winB, cntB, indB, ntotB)
        pltpu.async_copy(winB, out_h.at[pl.ds(tile_pbase + w1 * WIN, WIN)],
                         sem_o1)
        return carry
    lax.fori_loop(0, NWIN // 2, _pair, 0)

    pltpu.make_async_copy(winB, out_h.at[pl.ds(0, WIN)], sem_o1).wait()


@jax.jit
def _run(grid_flat, attr_positions, attr_head_ids, table_ext):
    return _encode(grid_flat, attr_positions, attr_head_ids, table_ext)


def kernel(block_type_grid, attr_positions, attr_head_ids, heads):
    Bt, Wt, Ht, Lt = block_type_grid.shape
    grid_flat = block_type_grid.reshape(-1)
    table_ext = jnp.concatenate(
        [heads.reshape(N_HEADS * N_BLOCKS, D),
         jnp.zeros((ZROWS, D), heads.dtype)], axis=0)
    out = _run(grid_flat, attr_positions, attr_head_ids, table_ext)
    return out.reshape(Bt, Wt, Ht, Lt, D)


# speedup vs baseline: 1.3042x; 1.0037x over previous
import functools

import jax
import jax.numpy as jnp
from jax import lax
from jax.experimental import pallas as pl
from jax.experimental.pallas import tpu as pltpu
from jax.experimental.pallas import tpu_sc as plsc

P_TOTAL = 131072
D = 256
N_HEADS = 8
N_BLOCKS = 256
N_ATTR = 65536
NC, NS = 2, 16
HALF = P_TOTAL // NC
P_TILE = HALF // NS
WIN = 128
NWIN = P_TILE // WIN
A_TILE = N_ATTR // NS
ZROWS = 128
TBL_ROWS = N_HEADS * N_BLOCKS + ZROWS
TRASH = 2048
CNT = N_HEADS * HALF
FSLOTS = 16
APC = 1024

_mesh = plsc.VectorSubcoreMesh(core_axis_name="c", subcore_axis_name="s")


@functools.partial(
    pl.kernel,
    mesh=_mesh,
    out_type=jax.ShapeDtypeStruct((P_TOTAL, D), jnp.float32),
    scratch_types=[
        pltpu.VMEM((APC // 128, 128), jnp.int32),
        pltpu.VMEM((128,), jnp.int32),
        pltpu.VMEM((1024,), jnp.int32),
        pltpu.VMEM((P_TILE,), jnp.int32),
        pltpu.VMEM((N_HEADS, WIN), jnp.int32),
        pltpu.VMEM((N_HEADS, WIN), jnp.int32),
        pltpu.VMEM((WIN,), jnp.int32),
        pltpu.VMEM((WIN,), jnp.int32),
        pltpu.VMEM((N_HEADS * 16,), jnp.int32),
        pltpu.VMEM((N_HEADS * 16,), jnp.int32),
        pltpu.VMEM((16,), jnp.int32),
        pltpu.VMEM((16,), jnp.int32),
        pltpu.VMEM((WIN, D), jnp.float32),
        pltpu.VMEM((WIN, D), jnp.float32),
        pltpu.VMEM((FSLOTS, D), jnp.float32),
        pltpu.VMEM((FSLOTS * 16,), jnp.int32),
        pltpu.VMEM((FSLOTS * 16,), jnp.float32),
        pltpu.VMEM((16,), jnp.int32),
        pltpu.VMEM_SHARED((CNT + TRASH,), jnp.int32),
        pltpu.SemaphoreType.DMA,
        pltpu.SemaphoreType.DMA,
        pltpu.SemaphoreType.DMA,
        pltpu.SemaphoreType.DMA,
        pltpu.SemaphoreType.DMA,
        pltpu.SemaphoreType.DMA,
        pltpu.SemaphoreType.DMA,
    ],
)
def _encode(grid_h, pos_h, head_h, tbl_h, out_h,
            sidx, ones_v, zeros_v, grid_all,
            cntA, cntB, idxA, idxB, indA, indB, ntotA, ntotB, winA, winB,
            rowfs, pploc, pscale, smcv, cnt_s,
            sem_a, sem_g0, sem_g1, sem_o0, sem_o1, sem_r, sem_c):
    c = lax.axis_index("c")
    s = lax.axis_index("s")
    half_base = c * HALF
    tile_pbase = half_base + s * P_TILE
    lanes = lax.broadcasted_iota(jnp.int32, (16,), 0)

    for g in range(128 // 16):
        ones_v[pl.ds(g * 16, 16)] = jnp.ones((16,), jnp.int32)

    def _zinit(g, carry):
        zeros_v[pl.ds(g * 16, 16)] = jnp.zeros((16,), jnp.int32)
        return carry
    lax.fori_loop(0, 1024 // 16, _zinit, 0)

    def _ziss(j, carry):
        pltpu.async_copy(zeros_v,
                         cnt_s.at[pl.ds(s * (CNT // NS) + j * 1024, 1024)],
                         sem_a)
        return carry
    lax.fori_loop(0, 32, _ziss, 0)

    def _zdrain(j, carry):
        pltpu.make_async_copy(
            zeros_v, cnt_s.at[pl.ds(s * (CNT // NS), 1024)], sem_a).wait()
        return carry
    lax.fori_loop(0, 32, _zdrain, 0)

    plsc.subcore_barrier()

    def _chunk(q, carry):
        pltpu.sync_copy(pos_h.at[pl.ds(s * A_TILE + q * APC, APC)],
                        grid_all.at[pl.ds(0, APC)])
        pltpu.sync_copy(head_h.at[pl.ds(s * A_TILE + q * APC, APC)],
                        grid_all.at[pl.ds(APC, APC)])

        def _bidx(g, carry2):
            p = grid_all[pl.ds(g * 16, 16)]
            h = grid_all[pl.ds(APC + g * 16, 16)]
            rel = p - half_base
            ok = (rel >= 0) & (rel < HALF)
            tgt = h * HALF + rel
            spread = s * 128 + ((q * APC + g * 16 + lanes) & 127)
            idx = jnp.where(ok, tgt, CNT + spread)
            row = g // 8
            col = (g % 8) * 16
            sidx[row, pl.ds(col, 16)] = idx
            return carry2
        lax.fori_loop(0, APC // 16, _bidx, 0)

        def _hiss(j, carry2):
            pltpu.async_copy(ones_v, cnt_s.at[sidx.at[j]], sem_a, add=True)
            return carry2
        lax.fori_loop(0, APC // 128, _hiss, 0)

        def _hdrain(j, carry2):
            pltpu.make_async_copy(ones_v, cnt_s.at[sidx.at[0]],
                                  sem_a).wait()
            return carry2
        lax.fori_loop(0, APC // 128, _hdrain, 0)
        return carry
    lax.fori_loop(0, A_TILE // APC, _chunk, 0)
    pltpu.sync_copy(grid_h.at[pl.ds(tile_pbase, P_TILE)], grid_all)

    plsc.subcore_barrier()

    def _plan(wq, cntb, idxb, indb, ntb):
        lbase = s * P_TILE + wq * WIN
        for h in range(N_HEADS):
            pltpu.async_copy(cnt_s.at[pl.ds(h * HALF + lbase, WIN)],
                             cntb.at[h], sem_c)
        for h in range(N_HEADS):
            pltpu.make_async_copy(cnt_s.at[pl.ds(0, WIN)],
                                  cntb.at[h], sem_c).wait()
        ind = [jnp.zeros((16,), jnp.int32) for _ in range(N_HEADS)]
        nfix = jnp.zeros((16,), jnp.int32)
        for g in range(WIN // 16):
            b = grid_all[pl.ds(wq * WIN + g * 16, 16)]
            first = N_HEADS * N_BLOCKS + g * 16 + lanes
            nsel = jnp.zeros((16,), jnp.int32)
            for h in range(N_HEADS):
                cv = cntb[h, pl.ds(g * 16, 16)]
                has = cv > 0
                isfirst = has & (nsel == 0)
                rowi = h * N_BLOCKS + b
                first = jnp.where(isfirst, rowi, first)
                bA = isfirst & (cv > 1)
                bB = has & (nsel > 0)
                ind[h] = (ind[h]
                          | jnp.where(bB, 1 << g, 0)
                          | jnp.where(bA, 1 << (8 + g), 0))
                nfix = (nfix + jnp.where(bA, 1, 0) + jnp.where(bB, 1, 0))
                nsel = nsel + jnp.where(has, 1, 0)
            idxb[pl.ds(g * 16, 16)] = first
        for h in range(N_HEADS):
            indb[pl.ds(h * 16, 16)] = ind[h]
        ntb[pl.ds(0, 16)] = nfix

    def _walkrec(w, cntb, indb, r16):
        smcv[pl.ds(0, 16)] = jnp.zeros((16,), jnp.int32)

        def _wh(h, carry):
            mv = indb[pl.ds(h * 16, 16)]
            for l in range(16):
                m0 = mv[l]

                @pl.when(m0 != 0)
                def _(m0=m0, l=l):
                    def _g2(g2, cc):
                        @pl.when(((m0 >> g2) & 1) != 0)
                        def _():
                            nf = smcv[pl.ds(0, 16)][0]

                            @pl.when((nf >= r16) & (nf < r16 + FSLOTS))
                            def _():
                                isA = g2 // 8
                                g = g2 & 7
                                b = grid_all[pl.ds(w * WIN + g * 16,
                                                   16)][l]
                                rowi = h * N_BLOCKS + b
                                cval = cntb[h, pl.ds(g * 16, 16)][l]
                                scale = (cval - isA).astype(jnp.float32)
                                slot = nf - r16
                                pploc[pl.ds(slot * 16, 16)] = (
                                    jnp.broadcast_to(g * 16 + l, (16,)))
                                pscale[pl.ds(slot * 16, 16)] = (
                                    jnp.broadcast_to(scale, (16,)))
                                pltpu.async_copy(
                                    tbl_h.at[rowi], rowfs.at[slot], sem_r)
                            smcv[pl.ds(0, 16)] = jnp.broadcast_to(
                                nf + 1, (16,))
                        return cc
                    lax.fori_loop(0, 16, _g2, 0)
            return carry
        lax.fori_loop(0, N_HEADS, _wh, 0)

    def _rdrain(n):
        def _rd(j, carry):
            pltpu.make_async_copy(tbl_h.at[0], rowfs.at[0], sem_r).wait()
            return carry
        lax.fori_loop(0, n, _rd, 0)

    def _fixups(w, winb, cntb, indb, ntb):
        nv = ntb[pl.ds(0, 16)]
        total = nv[0]
        for l in range(1, 16):
            total = total + nv[l]

        def _round(r, carry):
            r16 = r * FSLOTS
            nret = jnp.minimum(total - r16, FSLOTS)
            _walkrec(w, cntb, indb, r16)
            _rdrain(nret)

            def _apply(j, c2):
                ploc = pploc[pl.ds(j * 16, 16)][0]
                scv = jnp.broadcast_to(pscale[pl.ds(j * 16, 16)][0], (16,))

                def _fma(d, c3):
                    winb[ploc, pl.ds(d * 16, 16)] = (
                        winb[ploc, pl.ds(d * 16, 16)]
                        + scv * rowfs[j, pl.ds(d * 16, 16)])
                    return c3
                lax.fori_loop(0, D // 16, _fma, 0)
                return c2
            lax.fori_loop(0, nret, _apply, 0)
            return carry
        lax.fori_loop(0, (total + FSLOTS - 1) // FSLOTS, _round, 0)

    _plan(jnp.int32(0), cntA, idxA, indA, ntotA)
    pltpu.async_copy(tbl_h.at[idxA], winA, sem_g0)

    def _pair(k, carry):
        w0 = 2 * k
        w1 = w0 + 1

        _plan(w1, cntB, idxB, indB, ntotB)

        @pl.when(w0 >= 1)
        def _():
            pltpu.make_async_copy(winB, out_h.at[pl.ds(0, WIN)],
                                  sem_o1).wait()
        pltpu.async_copy(tbl_h.at[idxB], winB, sem_g1)
        pltpu.make_async_copy(tbl_h.at[idxA], winA, sem_g0).wait()
        _fixups(w0, winA, cntA, indA, ntotA)
        pltpu.async_copy(winA, out_h.at[pl.ds(tile_pbase + w0 * WIN, WIN)],
                         sem_o0)

        @pl.when(w1 + 1 < NWIN)
        def _():
            _plan(w1 + 1, cntA, idxA, indA, ntotA)
        pltpu.make_async_copy(winA, out_h.at[pl.ds(0, WIN)], sem_o0).wait()

        @pl.when(w1 + 1 < NWIN)
        def _():
            pltpu.async_copy(tbl_h.at[idxA], winA, sem_g0)
        pltpu.make_async_copy(tbl_h.at[idxB], winB, sem_g1).wait()
        _fixups(w1, winB, cntB, indB, ntotB)
        pltpu.async_copy(winB, out_h.at[pl.ds(tile_pbase + w1 * WIN, WIN)],
                         sem_o1)
        return carry
    lax.fori_loop(0, NWIN // 2, _pair, 0)

    pltpu.make_async_copy(winB, out_h.at[pl.ds(0, WIN)], sem_o1).wait()


@jax.jit
def _run(grid_flat, attr_positions, attr_head_ids, table_ext):
    return _encode(grid_flat, attr_positions, attr_head_ids, table_ext)


def kernel(block_type_grid, attr_positions, attr_head_ids, heads):
    Bt, Wt, Ht, Lt = block_type_grid.shape
    grid_flat = block_type_grid.reshape(-1)
    table_ext = jnp.concatenate(
        [heads.reshape(N_HEADS * N_BLOCKS, D),
         jnp.zeros((ZROWS, D), heads.dtype)], axis=0)
    out = _run(grid_flat, attr_positions, attr_head_ids, table_ext)
    return out.reshape(Bt, Wt, Ht, Lt, D)
